# Initial kernel scaffold; baseline (speedup 1.0000x reference)
import jax
import jax.numpy as jnp
from jax.experimental import pallas as pl


def kernel(s_feat, o_feat, os_edge_attr, ss_edge_attr, params, os_src, os_dst, ss_src, ss_dst):
    raise NotImplementedError("write your pallas kernel here")


# speedup vs baseline: 3.8847x; 3.8847x over previous
import functools

import jax
import jax.numpy as jnp
from jax import lax
from jax.experimental import pallas as pl
from jax.experimental.pallas import tpu as pltpu
from jax.experimental.pallas import tpu_sc as plsc

NSP = 10240
D = 128
EAW = 16
NC = 2
NSC = 16
SL = NSP // NSC
CH = 128
NEG = -3.0e38

f32 = jnp.float32
i32 = jnp.int32


def _mesh():
  return plsc.VectorSubcoreMesh(core_axis_name="c", subcore_axis_name="s",
                                num_cores=NC, num_subcores=NSC)


_SC_PARAMS = pltpu.CompilerParams(needs_layout_passes=False)


def _leaky(x):
  return jnp.where(x >= 0, x, 0.2 * x)


def _lane_iota():
  return lax.iota(i32, 16)


def _seg_max(arr, idx16, val16):
  def body(_):
    g = plsc.load_gather(arr, [idx16])
    need = val16 > g
    plsc.store_scatter(arr, [idx16], jnp.maximum(g, val16), mask=need)
    return jnp.any(need)
  lax.while_loop(lambda c: c, body, jnp.any(val16 > plsc.load_gather(arr, [idx16])))


def _seg_add(arr, aux, idx16, val16):
  lid = _lane_iota()
  def cond(pending):
    return jnp.any(pending)
  def body(pending):
    plsc.store_scatter(aux, [idx16], lid, mask=pending)
    win = (plsc.load_gather(aux, [idx16]) == lid) & pending
    g = plsc.load_gather(arr, [idx16])
    plsc.store_scatter(arr, [idx16], g + val16, mask=win)
    return pending & jnp.logical_not(win)
  lax.while_loop(cond, body, jnp.ones((16,), jnp.bool_))


def _fill1d(ref, n, value):
  def b(i, c):
    ref[pl.ds(i * 16, 16)] = jnp.full((16,), value, f32)
    return c
  lax.fori_loop(0, n // 16, b, 0)


def _merge_tiles(part_sh, macc, mtmp, moff, op):
  pltpu.sync_copy(part_sh.at[0, pl.ds(moff, SL)], macc)
  def mb(j, c):
    pltpu.sync_copy(part_sh.at[j, pl.ds(moff, SL)], mtmp)
    def vb(i, c2):
      a = macc[pl.ds(i * 16, 16)]
      b = mtmp[pl.ds(i * 16, 16)]
      macc[pl.ds(i * 16, 16)] = op(a, b)
      return c2
    lax.fori_loop(0, SL // 16, vb, 0)
    return c
  lax.fori_loop(1, NSC, mb, 0)


@functools.partial(jax.jit, static_argnames=("ne",))
def _k_soft(po, q, src, dst, ne):
  epw = ne // NSC
  chk = 2000
  nchk = epw // chk

  def body(po_h, q_h, src_h, dst_h, m_h, r_h, att_h,
           po_v, dst_v, e_v, acc_v, aux_v, m_v, r_v, srcc, qc, macc, mtmp,
           part_sh, m_sh, r_sh):
    cid = lax.axis_index("c")
    sid = lax.axis_index("s")
    base = sid * epw
    moff = sid * SL
    pltpu.sync_copy(po_h, po_v)
    pltpu.sync_copy(dst_h.at[pl.ds(base, epw)], dst_v)
    _fill1d(acc_v, NSP, NEG)

    def chunk(k, c):
      cb = base + k * chk
      pltpu.sync_copy(src_h.at[pl.ds(cb, chk)], srcc)
      pltpu.sync_copy(q_h.at[pl.ds(cb, chk)], qc)
      def vb(i, c2):
        s16 = srcc[pl.ds(i * 16, 16)]
        q16 = qc[pl.ds(i * 16, 16)]
        e16 = _leaky(plsc.load_gather(po_v, [s16]) + q16)
        off = k * chk + i * 16
        e_v[pl.ds(off, 16)] = e16
        d16 = dst_v[pl.ds(off, 16)]
        _seg_max(acc_v, d16, e16)
        return c2
      lax.fori_loop(0, chk // 16, vb, 0)
      return c
    lax.fori_loop(0, nchk, chunk, 0)

    pltpu.sync_copy(acc_v, part_sh.at[sid])
    plsc.subcore_barrier()
    _merge_tiles(part_sh, macc, mtmp, moff, jnp.maximum)
    pltpu.sync_copy(macc, m_sh.at[pl.ds(moff, SL)])
    @pl.when(cid == 0)
    def _():
      pltpu.sync_copy(macc, m_h.at[pl.ds(moff, SL)])
    plsc.subcore_barrier()
    pltpu.sync_copy(m_sh, m_v)
    _fill1d(acc_v, NSP, 0.0)

    def vb2(i, c):
      e16 = e_v[pl.ds(i * 16, 16)]
      d16 = dst_v[pl.ds(i * 16, 16)]
      ex = jnp.exp(e16 - plsc.load_gather(m_v, [d16]))
      _seg_add(acc_v, aux_v, d16, ex)
      return c
    lax.fori_loop(0, epw // 16, vb2, 0)

    pltpu.sync_copy(acc_v, part_sh.at[sid])
    plsc.subcore_barrier()
    _merge_tiles(part_sh, macc, mtmp, moff, jnp.add)
    def vb3(i, c):
      macc[pl.ds(i * 16, 16)] = 1.0 / (macc[pl.ds(i * 16, 16)] + 1e-9)
      return c
    lax.fori_loop(0, SL // 16, vb3, 0)
    pltpu.sync_copy(macc, r_sh.at[pl.ds(moff, SL)])
    @pl.when(cid == 0)
    def _():
      pltpu.sync_copy(macc, r_h.at[pl.ds(moff, SL)])
    plsc.subcore_barrier()
    pltpu.sync_copy(r_sh, r_v)

    def vb4(i, c):
      e16 = e_v[pl.ds(i * 16, 16)]
      d16 = dst_v[pl.ds(i * 16, 16)]
      mg = plsc.load_gather(m_v, [d16])
      rg = plsc.load_gather(r_v, [d16])
      e_v[pl.ds(i * 16, 16)] = jnp.exp(e16 - mg) * rg
      return c
    lax.fori_loop(0, epw // 16, vb4, 0)
    @pl.when(cid == 0)
    def _():
      pltpu.sync_copy(e_v, att_h.at[pl.ds(base, epw)])

  return pl.kernel(
      body,
      out_type=(jax.ShapeDtypeStruct((NSP,), f32),
                jax.ShapeDtypeStruct((NSP,), f32),
                jax.ShapeDtypeStruct((ne,), f32)),
      mesh=_mesh(),
      compiler_params=_SC_PARAMS,
      scratch_types=[
          pltpu.VMEM((NSP,), f32),
          pltpu.VMEM((epw,), i32),
          pltpu.VMEM((epw,), f32),
          pltpu.VMEM((NSP,), f32),
          pltpu.VMEM((NSP,), i32),
          pltpu.VMEM((NSP,), f32),
          pltpu.VMEM((NSP,), f32),
          pltpu.VMEM((chk,), i32),
          pltpu.VMEM((chk,), f32),
          pltpu.VMEM((SL,), f32),
          pltpu.VMEM((SL,), f32),
          pltpu.VMEM_SHARED((NSC, NSP), f32),
          pltpu.VMEM_SHARED((NSP,), f32),
          pltpu.VMEM_SHARED((NSP,), f32),
      ],
  )(po, q, src, dst)


@functools.partial(jax.jit, static_argnames=("ne",))
def _k_deg(src, ne):
  epw = ne // NSC

  def body(src_h, deg_h, src_v, acc_v, aux_v, macc, mtmp, part_sh):
    cid = lax.axis_index("c")
    sid = lax.axis_index("s")
    moff = sid * SL
    pltpu.sync_copy(src_h.at[pl.ds(sid * epw, epw)], src_v)
    _fill1d(acc_v, NSP, 0.0)
    ones = jnp.ones((16,), f32)
    def vb(i, c):
      s16 = src_v[pl.ds(i * 16, 16)]
      _seg_add(acc_v, aux_v, s16, ones)
      return c
    lax.fori_loop(0, epw // 16, vb, 0)
    pltpu.sync_copy(acc_v, part_sh.at[sid])
    plsc.subcore_barrier()
    _merge_tiles(part_sh, macc, mtmp, moff, jnp.add)
    @pl.when(cid == 0)
    def _():
      pltpu.sync_copy(macc, deg_h.at[pl.ds(moff, SL)])

  return pl.kernel(
      body,
      out_type=jax.ShapeDtypeStruct((NSP,), f32),
      mesh=_mesh(),
      compiler_params=_SC_PARAMS,
      scratch_types=[
          pltpu.VMEM((epw,), i32),
          pltpu.VMEM((NSP,), f32),
          pltpu.VMEM((NSP,), i32),
          pltpu.VMEM((SL,), f32),
          pltpu.VMEM((SL,), f32),
          pltpu.VMEM_SHARED((NSC, NSP), f32),
      ],
  )(src)


@functools.partial(jax.jit, static_argnames=("ne",))
def _k_heavy(src, dst, att, p_tab, ea2, ne):
  epc = ne // NC
  nch = epc // CH
  ipt = (nch + NSC - 1) // NSC

  def body(src_h, dst_h, att_h, p_h, ea2_h, aggp_h,
           src_c, dstw, att_c, rows, erows, sem, sem2, acc_sh):
    cid = lax.axis_index("c")
    sid = lax.axis_index("s")
    moff = sid * SL

    def zb(j, c):
      for f in range(D // 16):
        rows[j, pl.ds(f * 16, 16)] = jnp.zeros((16,), f32)
      return c
    lax.fori_loop(0, CH, zb, 0)
    for blk in range(SL // CH):
      pltpu.sync_copy(rows, acc_sh.at[pl.ds(moff + blk * CH, CH)])
    plsc.subcore_barrier()

    def chunk(i, c):
      k = sid + i * NSC
      @pl.when(k < nch)
      def _():
        ebase = cid * epc + k * CH
        pltpu.sync_copy(src_h.at[pl.ds(ebase, CH)], src_c)
        pltpu.sync_copy(dst_h.at[pl.ds(ebase, CH)], dstw.at[0])
        pltpu.sync_copy(att_h.at[pl.ds(ebase, CH)], att_c)
        cp = pltpu.async_copy(p_h.at[src_c], rows, sem)
        cp2 = pltpu.async_copy(ea2_h.at[pl.ds(ebase, CH)], erows, sem2)
        cp.wait()
        cp2.wait()
        def sb(j, c2):
          ab16 = plsc.load_gather(att_c, [jnp.full((16,), j, i32)])
          for f in range(D // 16):
            rows[j, pl.ds(f * 16, 16)] = (
                rows[j, pl.ds(f * 16, 16)] + erows[j, pl.ds(f * 16, 16)]) * ab16
          return c2
        lax.fori_loop(0, CH, sb, 0)
        pltpu.sync_copy(rows, acc_sh.at[dstw.at[0]], add=True)
      return c
    lax.fori_loop(0, ipt, chunk, 0)

    plsc.subcore_barrier()
    pltpu.sync_copy(acc_sh.at[pl.ds(moff, SL)], aggp_h.at[cid, pl.ds(moff, SL)])

  return pl.kernel(
      body,
      out_type=jax.ShapeDtypeStruct((NC, NSP, D), f32),
      mesh=_mesh(),
      compiler_params=_SC_PARAMS,
      scratch_types=[
          pltpu.VMEM((CH,), i32),
          pltpu.VMEM((1, CH), i32),
          pltpu.VMEM((CH,), f32),
          pltpu.VMEM((CH, D), f32),
          pltpu.VMEM((CH, D), f32),
          pltpu.SemaphoreType.DMA,
          pltpu.SemaphoreType.DMA,
          pltpu.VMEM_SHARED((NSP, D), f32),
      ],
  )(src, dst, att, p_tab, ea2)


@functools.partial(jax.jit, static_argnames=("ne",))
def _k_msg(gidx, sidx, tab, ne):
  epc = ne // NC
  nch = epc // CH
  ipt = (nch + NSC - 1) // NSC

  def body(g_h, s_h, tab_h, out_h, g_c, sw, rows, sem, acc_sh):
    cid = lax.axis_index("c")
    sid = lax.axis_index("s")
    moff = sid * SL
    def zb(j, c):
      for f in range(D // 16):
        rows[j, pl.ds(f * 16, 16)] = jnp.zeros((16,), f32)
      return c
    lax.fori_loop(0, CH, zb, 0)
    for blk in range(SL // CH):
      pltpu.sync_copy(rows, acc_sh.at[pl.ds(moff + blk * CH, CH)])
    plsc.subcore_barrier()

    def chunk(i, c):
      k = sid + i * NSC
      @pl.when(k < nch)
      def _():
        ebase = cid * epc + k * CH
        pltpu.sync_copy(g_h.at[pl.ds(ebase, CH)], g_c)
        pltpu.sync_copy(s_h.at[pl.ds(ebase, CH)], sw.at[0])
        pltpu.async_copy(tab_h.at[g_c], rows, sem).wait()
        pltpu.sync_copy(rows, acc_sh.at[sw.at[0]], add=True)
      return c
    lax.fori_loop(0, ipt, chunk, 0)

    plsc.subcore_barrier()
    pltpu.sync_copy(acc_sh.at[pl.ds(moff, SL)], out_h.at[cid, pl.ds(moff, SL)])

  return pl.kernel(
      body,
      out_type=jax.ShapeDtypeStruct((NC, NSP, D), f32),
      mesh=_mesh(),
      compiler_params=_SC_PARAMS,
      scratch_types=[
          pltpu.VMEM((CH,), i32),
          pltpu.VMEM((1, CH), i32),
          pltpu.VMEM((CH, D), f32),
          pltpu.SemaphoreType.DMA,
          pltpu.VMEM_SHARED((NSP, D), f32),
      ],
  )(gidx, sidx, tab)


@functools.partial(jax.jit, static_argnames=("ne",))
def _k_logits(src, dst, s_tab, o_tab, ne):
  epc = ne // NC
  nch = epc // CH
  ipt = (nch + NSC - 1) // NSC

  def body(src_h, dst_h, s_h, o_h, out_h, src_c, dst_c, srows, orows,
           lg_c, sem1, sem2):
    cid = lax.axis_index("c")
    sid = lax.axis_index("s")
    def chunk(i, c):
      k = sid + i * NSC
      @pl.when(k < nch)
      def _():
        ebase = cid * epc + k * CH
        pltpu.sync_copy(src_h.at[pl.ds(ebase, CH)], src_c)
        pltpu.sync_copy(dst_h.at[pl.ds(ebase, CH)], dst_c)
        cp1 = pltpu.async_copy(o_h.at[src_c], orows, sem1)
        cp2 = pltpu.async_copy(s_h.at[dst_c], srows, sem2)
        cp1.wait()
        cp2.wait()
        lid = _lane_iota()
        def gb(g, c2):
          def jb(jj, out16):
            j = g * 16 + jj
            acc = srows[j, pl.ds(0, 16)] * orows[j, pl.ds(0, 16)]
            for f in range(1, D // 16):
              acc = acc + srows[j, pl.ds(f * 16, 16)] * orows[j, pl.ds(f * 16, 16)]
            dot = jnp.sum(acc)
            return jnp.where(lid == jj, dot, out16)
          out16 = lax.fori_loop(0, 16, jb, jnp.zeros((16,), f32))
          lg_c[pl.ds(g * 16, 16)] = out16
          return c2
        lax.fori_loop(0, CH // 16, gb, 0)
        pltpu.sync_copy(lg_c, out_h.at[pl.ds(ebase, CH)])
      return c
    lax.fori_loop(0, ipt, chunk, 0)

  return pl.kernel(
      body,
      out_type=jax.ShapeDtypeStruct((ne,), f32),
      mesh=_mesh(),
      compiler_params=_SC_PARAMS,
      scratch_types=[
          pltpu.VMEM((CH,), i32),
          pltpu.VMEM((CH,), i32),
          pltpu.VMEM((CH, D), f32),
          pltpu.VMEM((CH, D), f32),
          pltpu.VMEM((CH,), f32),
          pltpu.SemaphoreType.DMA,
          pltpu.SemaphoreType.DMA,
      ],
  )(src, dst, s_tab, o_tab)


@functools.partial(jax.jit, static_argnames=("ne",))
def _k_delta(src, td_tab, eag_flat, ne):
  epc = ne // NC
  nch = epc // CH
  ipt = (nch + NSC - 1) // NSC

  def body(src_h, td_h, eag_h, out_h, src_c, rows, eagv, sem):
    cid = lax.axis_index("c")
    sid = lax.axis_index("s")
    def chunk(i, c):
      k = sid + i * NSC
      @pl.when(k < nch)
      def _():
        ebase = cid * epc + k * CH
        pltpu.sync_copy(src_h.at[pl.ds(ebase, CH)], src_c)
        cp = pltpu.async_copy(td_h.at[src_c], rows, sem)
        pltpu.sync_copy(eag_h.at[pl.ds(ebase * EAW, CH * EAW)], eagv)
        cp.wait()
        def jb(j, c2):
          eagv[pl.ds(j * EAW, 16)] = rows[j, pl.ds(0, 16)] + eagv[pl.ds(j * EAW, 16)]
          return c2
        lax.fori_loop(0, CH, jb, 0)
        pltpu.sync_copy(eagv, out_h.at[pl.ds(ebase * EAW, CH * EAW)])
      return c
    lax.fori_loop(0, ipt, chunk, 0)

  return pl.kernel(
      body,
      out_type=jax.ShapeDtypeStruct((ne * EAW,), f32),
      mesh=_mesh(),
      compiler_params=_SC_PARAMS,
      scratch_types=[
          pltpu.VMEM((CH,), i32),
          pltpu.VMEM((CH, D), f32),
          pltpu.VMEM((CH * EAW,), f32),
          pltpu.SemaphoreType.DMA,
      ],
  )(src, td_tab, eag_flat)


def _mm(x, w, bias=None, relu=False, x2=None):
  m, kk = x.shape
  n = w.shape[1]
  bm = 512
  grid = m // bm
  have_b = bias is not None
  have_x2 = x2 is not None

  def body(*refs):
    idx = 0
    x_ref = refs[idx]; idx += 1
    if have_x2:
      x2_ref = refs[idx]; idx += 1
    w_ref = refs[idx]; idx += 1
    if have_b:
      b_ref = refs[idx]; idx += 1
    o_ref = refs[idx]
    xv = x_ref[...]
    if have_x2:
      xv = xv + x2_ref[...]
    acc = jnp.dot(xv, w_ref[...], preferred_element_type=f32)
    if have_b:
      acc = acc + b_ref[...]
    if relu:
      acc = jnp.maximum(acc, 0.0)
    o_ref[...] = acc

  in_specs = [pl.BlockSpec((bm, kk), lambda i: (i, 0))]
  args = [x]
  if have_x2:
    in_specs.append(pl.BlockSpec((bm, kk), lambda i: (i, 0)))
    args.append(x2)
  in_specs.append(pl.BlockSpec((kk, n), lambda i: (0, 0)))
  args.append(w)
  if have_b:
    in_specs.append(pl.BlockSpec((1, n), lambda i: (0, 0)))
    args.append(bias.reshape(1, n))
  return pl.pallas_call(
      body, grid=(grid,), in_specs=in_specs,
      out_specs=pl.BlockSpec((bm, n), lambda i: (i, 0)),
      out_shape=jax.ShapeDtypeStruct((m, n), f32))(*args)


def _mv(x, w, c):
  m, kk = x.shape
  bm = 512
  grid = m // bm

  def body(x_ref, w_ref, c_ref, o_ref):
    o_ref[...] = jnp.sum(x_ref[...] * w_ref[...], axis=1) + c_ref[...]

  return pl.pallas_call(
      body, grid=(grid,),
      in_specs=[pl.BlockSpec((bm, kk), lambda i: (i, 0)),
                pl.BlockSpec((1, kk), lambda i: (0, 0)),
                pl.BlockSpec((1,), lambda i: (0,))],
      out_specs=pl.BlockSpec((bm,), lambda i: (i,)),
      out_shape=jax.ShapeDtypeStruct((m,), f32))(
          x, w.reshape(1, kk), jnp.asarray(c, f32).reshape(1))


def _combine_s(a0, a1, a2, a3, r_os, r_ss, b_os, b_ss, relu):
  bm = 512
  grid = NSP // bm

  def body(a0r, a1r, a2r, a3r, ror, rsr, bor, bsr, o_ref):
    satt_os = 1.0 - 1e-9 * ror[...]
    satt_ss = 1.0 - 1e-9 * rsr[...]
    acc = (a0r[...] + a1r[...] + a2r[...] + a3r[...]
           + satt_os[:, None] * bor[...] + satt_ss[:, None] * bsr[...])
    if relu:
      acc = jnp.maximum(acc, 0.0)
    o_ref[...] = acc

  bs2 = pl.BlockSpec((bm, D), lambda i: (i, 0))
  bs1 = pl.BlockSpec((bm,), lambda i: (i,))
  bsb = pl.BlockSpec((1, D), lambda i: (0, 0))
  return pl.pallas_call(
      body, grid=(grid,),
      in_specs=[bs2, bs2, bs2, bs2, bs1, bs1, bsb, bsb],
      out_specs=bs2,
      out_shape=jax.ShapeDtypeStruct((NSP, D), f32))(
          a0, a1, a2, a3, r_os, r_ss,
          b_os.reshape(1, D), b_ss.reshape(1, D))


def _combine_o(o_mm, msg_mm, deg, b_o, b_so, relu):
  bm = 512
  grid = NSP // bm

  def body(omr, mmr, dgr, bor, bsr, o_ref):
    dg = dgr[...]
    acc = omr[...] + bor[...] + (mmr[...] + dg[:, None] * bsr[...]) / (dg[:, None] + 1.0)
    if relu:
      acc = jnp.maximum(acc, 0.0)
    o_ref[...] = acc

  bs2 = pl.BlockSpec((bm, D), lambda i: (i, 0))
  bs1 = pl.BlockSpec((bm,), lambda i: (i,))
  bsb = pl.BlockSpec((1, D), lambda i: (0, 0))
  return pl.pallas_call(
      body, grid=(grid,),
      in_specs=[bs2, bs2, bs1, bsb, bsb],
      out_specs=bs2,
      out_shape=jax.ShapeDtypeStruct((NSP, D), f32))(
          o_mm, msg_mm, deg, b_o.reshape(1, D), b_so.reshape(1, D))


def kernel(s_feat, o_feat, os_edge_attr, ss_edge_attr, params,
           os_src, os_dst, ss_src, ss_dst):
  ns, _ = s_feat.shape
  no, _ = o_feat.shape
  ne = os_src.shape[0]
  n_layers = len(params)

  pad_n = lambda x: jnp.pad(x, ((0, NSP - x.shape[0]), (0, 0)))
  s_cur = pad_n(s_feat.astype(f32))
  o_cur = pad_n(o_feat.astype(f32))
  os_src = os_src.astype(i32)
  os_dst = os_dst.astype(i32)
  ss_src = ss_src.astype(i32)
  ss_dst = ss_dst.astype(i32)
  os_ea = os_edge_attr.astype(f32)
  ss_ea = ss_edge_attr.astype(f32)

  deg = _k_deg(os_src, ne=ne)

  s_hid = o_hid = delta16 = None
  for li, p in enumerate(params):
    od = p['W_o'].shape[0]
    sd = p['W_so'].shape[0]
    w_os_top, w_os_bot = p['W_os'][:od], p['W_os'][od:]
    w_ss_top, w_ss_bot = p['W_ss'][:sd], p['W_ss'][sd:]
    wq_os = w_os_bot @ p['a_os']
    wq_ss = w_ss_bot @ p['a_ss']
    c_os = jnp.dot(p['b_os'], p['a_os'])
    c_ss = jnp.dot(p['b_ss'], p['a_ss'])

    p_o = _mm(o_cur, w_os_top)
    p_s = _mm(s_cur, w_ss_top)
    po = _mv(p_o, p['a_os'], 0.0)
    ps = _mv(p_s, p['a_ss'], 0.0)
    q_os = _mv(os_ea, wq_os, c_os)
    q_ss = _mv(ss_ea, wq_ss, c_ss)

    m_os, r_os, att_os = _k_soft(po, q_os, os_src, os_dst, ne=ne)
    m_ss, r_ss, att_ss = _k_soft(ps, q_ss, ss_src, ss_dst, ne=ne)

    ea2_os = _mm(os_ea, w_os_bot)
    ea2_ss = _mm(ss_ea, w_ss_bot)
    aggp_os = _k_heavy(os_src, os_dst, att_os, p_o, ea2_os, ne=ne)
    aggp_ss = _k_heavy(ss_src, ss_dst, att_ss, p_s, ea2_ss, ne=ne)
    msgp = _k_msg(os_dst, os_src, s_cur, ne=ne)

    msg_mm = _mm(msgp[0], p['W_so'], x2=msgp[1])
    o_mm = _mm(o_cur, p['W_o'])

    last = li == n_layers - 1
    s_hid = _combine_s(aggp_os[0], aggp_os[1], aggp_ss[0], aggp_ss[1],
                       r_os, r_ss, p['b_os'], p['b_ss'],
                       relu=not last)
    o_hid = _combine_o(o_mm, msg_mm, deg, p['b_o'], p['b_so'], relu=not last)

    if last:
      wd128 = jnp.pad(p['W_delta'], ((0, 0), (0, D - p['W_delta'].shape[1])))
      td128 = _mm(p_s, wd128)
      g16 = w_ss_bot @ wd128[:, :EAW]
      cvec16 = p['b_ss'] @ wd128[:, :EAW] + jnp.pad(
          p['b_delta'], (0, EAW - p['b_delta'].shape[0]))
      eag = _mm(ss_ea, g16, bias=cvec16)
      delta16 = _k_delta(ss_src, td128, eag.reshape(-1), ne=ne).reshape(ne, EAW)

    s_cur, o_cur = s_hid, o_hid

  logits = _k_logits(os_src, os_dst, s_hid, o_hid, ne=ne)
  return (logits, delta16[:, :p['W_delta'].shape[1]])


# speedup vs baseline: 4.0173x; 1.0341x over previous
import functools

import jax
import jax.numpy as jnp
from jax import lax
from jax.experimental import pallas as pl
from jax.experimental.pallas import tpu as pltpu
from jax.experimental.pallas import tpu_sc as plsc

NSP = 10240
D = 128
EAW = 16
NC = 2
NSC = 16
SL = NSP // NSC
CH = 128
NEG = -3.0e38

f32 = jnp.float32
i32 = jnp.int32


def _mesh():
  return plsc.VectorSubcoreMesh(core_axis_name="c", subcore_axis_name="s",
                                num_cores=NC, num_subcores=NSC)


_SC_PARAMS = pltpu.CompilerParams(needs_layout_passes=False)


def _leaky(x):
  return jnp.where(x >= 0, x, 0.2 * x)


def _lane_iota():
  return lax.iota(i32, 16)


def _seg_max(arr, idx16, val16):
  def body(_):
    g = plsc.load_gather(arr, [idx16])
    need = val16 > g
    plsc.store_scatter(arr, [idx16], jnp.maximum(g, val16), mask=need)
    return jnp.any(need)
  lax.while_loop(lambda c: c, body, jnp.any(val16 > plsc.load_gather(arr, [idx16])))


def _seg_add(arr, aux, idx16, val16):
  lid = _lane_iota()
  def cond(pending):
    return jnp.any(pending)
  def body(pending):
    plsc.store_scatter(aux, [idx16], lid, mask=pending)
    win = (plsc.load_gather(aux, [idx16]) == lid) & pending
    g = plsc.load_gather(arr, [idx16])
    plsc.store_scatter(arr, [idx16], g + val16, mask=win)
    return pending & jnp.logical_not(win)
  lax.while_loop(cond, body, jnp.ones((16,), jnp.bool_))


def _fill1d(ref, n, value):
  def b(i, c):
    ref[pl.ds(i * 16, 16)] = jnp.full((16,), value, f32)
    return c
  lax.fori_loop(0, n // 16, b, 0)


def _merge_tiles(part_sh, macc, mtmp, moff, op):
  pltpu.sync_copy(part_sh.at[0, pl.ds(moff, SL)], macc)
  def mb(j, c):
    pltpu.sync_copy(part_sh.at[j, pl.ds(moff, SL)], mtmp)
    def vb(i, c2):
      a = macc[pl.ds(i * 16, 16)]
      b = mtmp[pl.ds(i * 16, 16)]
      macc[pl.ds(i * 16, 16)] = op(a, b)
      return c2
    lax.fori_loop(0, SL // 16, vb, 0)
    return c
  lax.fori_loop(1, NSC, mb, 0)


@functools.partial(jax.jit, static_argnames=("ne",))
def _k_soft(po, q, src, dst, ne):
  epw = ne // NSC
  chk = 2000
  nchk = epw // chk

  def body(po_h, q_h, src_h, dst_h, m_h, r_h, att_h,
           po_v, dst_v, e_v, acc_v, aux_v, m_v, r_v, srcc, qc, macc, mtmp,
           part_sh, m_sh, r_sh):
    cid = lax.axis_index("c")
    sid = lax.axis_index("s")
    base = sid * epw
    moff = sid * SL
    pltpu.sync_copy(po_h, po_v)
    pltpu.sync_copy(dst_h.at[pl.ds(base, epw)], dst_v)
    _fill1d(acc_v, NSP, NEG)

    def chunk(k, c):
      cb = base + k * chk
      pltpu.sync_copy(src_h.at[pl.ds(cb, chk)], srcc)
      pltpu.sync_copy(q_h.at[pl.ds(cb, chk)], qc)
      def vb(i, c2):
        s16 = srcc[pl.ds(i * 16, 16)]
        q16 = qc[pl.ds(i * 16, 16)]
        e16 = _leaky(plsc.load_gather(po_v, [s16]) + q16)
        off = k * chk + i * 16
        e_v[pl.ds(off, 16)] = e16
        d16 = dst_v[pl.ds(off, 16)]
        _seg_max(acc_v, d16, e16)
        return c2
      lax.fori_loop(0, chk // 16, vb, 0)
      return c
    lax.fori_loop(0, nchk, chunk, 0)

    pltpu.sync_copy(acc_v, part_sh.at[sid])
    plsc.subcore_barrier()
    _merge_tiles(part_sh, macc, mtmp, moff, jnp.maximum)
    pltpu.sync_copy(macc, m_sh.at[pl.ds(moff, SL)])
    @pl.when(cid == 0)
    def _():
      pltpu.sync_copy(macc, m_h.at[pl.ds(moff, SL)])
    plsc.subcore_barrier()
    pltpu.sync_copy(m_sh, m_v)
    _fill1d(acc_v, NSP, 0.0)

    def vb2(i, c):
      e16 = e_v[pl.ds(i * 16, 16)]
      d16 = dst_v[pl.ds(i * 16, 16)]
      ex = jnp.exp(e16 - plsc.load_gather(m_v, [d16]))
      _seg_add(acc_v, aux_v, d16, ex)
      return c
    lax.fori_loop(0, epw // 16, vb2, 0)

    pltpu.sync_copy(acc_v, part_sh.at[sid])
    plsc.subcore_barrier()
    _merge_tiles(part_sh, macc, mtmp, moff, jnp.add)
    def vb3(i, c):
      macc[pl.ds(i * 16, 16)] = 1.0 / (macc[pl.ds(i * 16, 16)] + 1e-9)
      return c
    lax.fori_loop(0, SL // 16, vb3, 0)
    pltpu.sync_copy(macc, r_sh.at[pl.ds(moff, SL)])
    @pl.when(cid == 0)
    def _():
      pltpu.sync_copy(macc, r_h.at[pl.ds(moff, SL)])
    plsc.subcore_barrier()
    pltpu.sync_copy(r_sh, r_v)

    def vb4(i, c):
      e16 = e_v[pl.ds(i * 16, 16)]
      d16 = dst_v[pl.ds(i * 16, 16)]
      mg = plsc.load_gather(m_v, [d16])
      rg = plsc.load_gather(r_v, [d16])
      e_v[pl.ds(i * 16, 16)] = jnp.exp(e16 - mg) * rg
      return c
    lax.fori_loop(0, epw // 16, vb4, 0)
    @pl.when(cid == 0)
    def _():
      pltpu.sync_copy(e_v, att_h.at[pl.ds(base, epw)])

  return pl.kernel(
      body,
      out_type=(jax.ShapeDtypeStruct((NSP,), f32),
                jax.ShapeDtypeStruct((NSP,), f32),
                jax.ShapeDtypeStruct((ne,), f32)),
      mesh=_mesh(),
      compiler_params=_SC_PARAMS,
      scratch_types=[
          pltpu.VMEM((NSP,), f32),
          pltpu.VMEM((epw,), i32),
          pltpu.VMEM((epw,), f32),
          pltpu.VMEM((NSP,), f32),
          pltpu.VMEM((NSP,), i32),
          pltpu.VMEM((NSP,), f32),
          pltpu.VMEM((NSP,), f32),
          pltpu.VMEM((chk,), i32),
          pltpu.VMEM((chk,), f32),
          pltpu.VMEM((SL,), f32),
          pltpu.VMEM((SL,), f32),
          pltpu.VMEM_SHARED((NSC, NSP), f32),
          pltpu.VMEM_SHARED((NSP,), f32),
          pltpu.VMEM_SHARED((NSP,), f32),
      ],
  )(po, q, src, dst)


@functools.partial(jax.jit, static_argnames=("ne",))
def _k_deg(src, ne):
  epw = ne // NSC

  def body(src_h, deg_h, src_v, acc_v, aux_v, macc, mtmp, part_sh):
    cid = lax.axis_index("c")
    sid = lax.axis_index("s")
    moff = sid * SL
    pltpu.sync_copy(src_h.at[pl.ds(sid * epw, epw)], src_v)
    _fill1d(acc_v, NSP, 0.0)
    ones = jnp.ones((16,), f32)
    def vb(i, c):
      s16 = src_v[pl.ds(i * 16, 16)]
      _seg_add(acc_v, aux_v, s16, ones)
      return c
    lax.fori_loop(0, epw // 16, vb, 0)
    pltpu.sync_copy(acc_v, part_sh.at[sid])
    plsc.subcore_barrier()
    _merge_tiles(part_sh, macc, mtmp, moff, jnp.add)
    @pl.when(cid == 0)
    def _():
      pltpu.sync_copy(macc, deg_h.at[pl.ds(moff, SL)])

  return pl.kernel(
      body,
      out_type=jax.ShapeDtypeStruct((NSP,), f32),
      mesh=_mesh(),
      compiler_params=_SC_PARAMS,
      scratch_types=[
          pltpu.VMEM((epw,), i32),
          pltpu.VMEM((NSP,), f32),
          pltpu.VMEM((NSP,), i32),
          pltpu.VMEM((SL,), f32),
          pltpu.VMEM((SL,), f32),
          pltpu.VMEM_SHARED((NSC, NSP), f32),
      ],
  )(src)


@functools.partial(jax.jit, static_argnames=("ne",))
def _k_heavy(src, dst, att, p_tab, ea2, ne):
  chh = 64
  epc = ne // NC
  nch = epc // chh
  ipt = (nch + NSC - 1) // NSC
  npair = (ipt + 1) // 2

  def body(src_h, dst_h, att_h, p_h, ea2_h, aggp_h, *scr):
    (src_c, dstw, att_c, rows, erows, sems, esems, acc_sh) = (
        scr[0:2], scr[2:4], scr[4:6], scr[6:8], scr[8:10], scr[10:12],
        scr[12:14], scr[14])
    cid = lax.axis_index("c")
    sid = lax.axis_index("s")
    moff = sid * SL

    def zb(j, c):
      for f in range(D // 16):
        rows[0][j, pl.ds(f * 16, 16)] = jnp.zeros((16,), f32)
      return c
    lax.fori_loop(0, chh, zb, 0)
    for blk in range(SL // chh):
      pltpu.sync_copy(rows[0], acc_sh.at[pl.ds(moff + blk * chh, chh)])
    plsc.subcore_barrier()

    def issue(b, k):
      @pl.when(k < nch)
      def _():
        ebase = cid * epc + k * chh
        pltpu.sync_copy(src_h.at[pl.ds(ebase, chh)], src_c[b])
        pltpu.sync_copy(dst_h.at[pl.ds(ebase, chh)], dstw[b].at[0])
        pltpu.sync_copy(att_h.at[pl.ds(ebase, chh)], att_c[b])
        pltpu.async_copy(p_h.at[src_c[b]], rows[b], sems[b])
        pltpu.async_copy(ea2_h.at[pl.ds(ebase, chh)], erows[b], esems[b])

    def finish(b, k):
      @pl.when(k < nch)
      def _():
        pltpu.make_async_copy(p_h.at[src_c[b]], rows[b], sems[b]).wait()
        pltpu.make_async_copy(ea2_h.at[pl.ds(0, chh)], erows[b], esems[b]).wait()
        def sb(j, c2):
          ab16 = plsc.load_gather(att_c[b], [jnp.full((16,), j, i32)])
          for f in range(D // 16):
            rows[b][j, pl.ds(f * 16, 16)] = (
                rows[b][j, pl.ds(f * 16, 16)]
                + erows[b][j, pl.ds(f * 16, 16)]) * ab16
          return c2
        lax.fori_loop(0, chh, sb, 0)
        pltpu.sync_copy(rows[b], acc_sh.at[dstw[b].at[0]], add=True)

    issue(0, sid)
    def pair(i, c):
      k0 = sid + (2 * i) * NSC
      k1 = sid + (2 * i + 1) * NSC
      issue(1, k1)
      finish(0, k0)
      issue(0, sid + (2 * i + 2) * NSC)
      finish(1, k1)
      return c
    lax.fori_loop(0, npair, pair, 0)

    plsc.subcore_barrier()
    pltpu.sync_copy(acc_sh.at[pl.ds(moff, SL)], aggp_h.at[cid, pl.ds(moff, SL)])

  return pl.kernel(
      body,
      out_type=jax.ShapeDtypeStruct((NC, NSP, D), f32),
      mesh=_mesh(),
      compiler_params=_SC_PARAMS,
      scratch_types=[
          pltpu.VMEM((chh,), i32),
          pltpu.VMEM((chh,), i32),
          pltpu.VMEM((1, chh), i32),
          pltpu.VMEM((1, chh), i32),
          pltpu.VMEM((chh,), f32),
          pltpu.VMEM((chh,), f32),
          pltpu.VMEM((chh, D), f32),
          pltpu.VMEM((chh, D), f32),
          pltpu.VMEM((chh, D), f32),
          pltpu.VMEM((chh, D), f32),
          pltpu.SemaphoreType.DMA,
          pltpu.SemaphoreType.DMA,
          pltpu.SemaphoreType.DMA,
          pltpu.SemaphoreType.DMA,
          pltpu.VMEM_SHARED((NSP, D), f32),
      ],
  )(src, dst, att, p_tab, ea2)


@functools.partial(jax.jit, static_argnames=("ne",))
def _k_msg(gidx, sidx, tab, ne):
  epc = ne // NC
  nch = epc // CH
  ipt = (nch + NSC - 1) // NSC

  def body(g_h, s_h, tab_h, out_h, g_c, sw, rows, sem, acc_sh):
    cid = lax.axis_index("c")
    sid = lax.axis_index("s")
    moff = sid * SL
    def zb(j, c):
      for f in range(D // 16):
        rows[j, pl.ds(f * 16, 16)] = jnp.zeros((16,), f32)
      return c
    lax.fori_loop(0, CH, zb, 0)
    for blk in range(SL // CH):
      pltpu.sync_copy(rows, acc_sh.at[pl.ds(moff + blk * CH, CH)])
    plsc.subcore_barrier()

    def chunk(i, c):
      k = sid + i * NSC
      @pl.when(k < nch)
      def _():
        ebase = cid * epc + k * CH
        pltpu.sync_copy(g_h.at[pl.ds(ebase, CH)], g_c)
        pltpu.sync_copy(s_h.at[pl.ds(ebase, CH)], sw.at[0])
        pltpu.async_copy(tab_h.at[g_c], rows, sem).wait()
        pltpu.sync_copy(rows, acc_sh.at[sw.at[0]], add=True)
      return c
    lax.fori_loop(0, ipt, chunk, 0)

    plsc.subcore_barrier()
    pltpu.sync_copy(acc_sh.at[pl.ds(moff, SL)], out_h.at[cid, pl.ds(moff, SL)])

  return pl.kernel(
      body,
      out_type=jax.ShapeDtypeStruct((NC, NSP, D), f32),
      mesh=_mesh(),
      compiler_params=_SC_PARAMS,
      scratch_types=[
          pltpu.VMEM((CH,), i32),
          pltpu.VMEM((1, CH), i32),
          pltpu.VMEM((CH, D), f32),
          pltpu.SemaphoreType.DMA,
          pltpu.VMEM_SHARED((NSP, D), f32),
      ],
  )(gidx, sidx, tab)


@functools.partial(jax.jit, static_argnames=("ne",))
def _k_logits(src, dst, s_tab, o_tab, ne):
  epc = ne // NC
  nch = epc // CH
  ipt = (nch + NSC - 1) // NSC

  def body(src_h, dst_h, s_h, o_h, out_h, src_c, dst_c, srows, orows,
           lg_c, sem1, sem2):
    cid = lax.axis_index("c")
    sid = lax.axis_index("s")
    def chunk(i, c):
      k = sid + i * NSC
      @pl.when(k < nch)
      def _():
        ebase = cid * epc + k * CH
        pltpu.sync_copy(src_h.at[pl.ds(ebase, CH)], src_c)
        pltpu.sync_copy(dst_h.at[pl.ds(ebase, CH)], dst_c)
        cp1 = pltpu.async_copy(o_h.at[src_c], orows, sem1)
        cp2 = pltpu.async_copy(s_h.at[dst_c], srows, sem2)
        cp1.wait()
        cp2.wait()
        lid = _lane_iota()
        def gb(g, c2):
          def jb(jj, out16):
            j = g * 16 + jj
            acc = srows[j, pl.ds(0, 16)] * orows[j, pl.ds(0, 16)]
            for f in range(1, D // 16):
              acc = acc + srows[j, pl.ds(f * 16, 16)] * orows[j, pl.ds(f * 16, 16)]
            dot = jnp.sum(acc)
            return jnp.where(lid == jj, dot, out16)
          out16 = lax.fori_loop(0, 16, jb, jnp.zeros((16,), f32))
          lg_c[pl.ds(g * 16, 16)] = out16
          return c2
        lax.fori_loop(0, CH // 16, gb, 0)
        pltpu.sync_copy(lg_c, out_h.at[pl.ds(ebase, CH)])
      return c
    lax.fori_loop(0, ipt, chunk, 0)

  return pl.kernel(
      body,
      out_type=jax.ShapeDtypeStruct((ne,), f32),
      mesh=_mesh(),
      compiler_params=_SC_PARAMS,
      scratch_types=[
          pltpu.VMEM((CH,), i32),
          pltpu.VMEM((CH,), i32),
          pltpu.VMEM((CH, D), f32),
          pltpu.VMEM((CH, D), f32),
          pltpu.VMEM((CH,), f32),
          pltpu.SemaphoreType.DMA,
          pltpu.SemaphoreType.DMA,
      ],
  )(src, dst, s_tab, o_tab)


@functools.partial(jax.jit, static_argnames=("ne",))
def _k_delta(src, td_tab, eag_flat, ne):
  epc = ne // NC
  nch = epc // CH
  ipt = (nch + NSC - 1) // NSC

  def body(src_h, td_h, eag_h, out_h, src_c, rows, eagv, sem):
    cid = lax.axis_index("c")
    sid = lax.axis_index("s")
    def chunk(i, c):
      k = sid + i * NSC
      @pl.when(k < nch)
      def _():
        ebase = cid * epc + k * CH
        pltpu.sync_copy(src_h.at[pl.ds(ebase, CH)], src_c)
        cp = pltpu.async_copy(td_h.at[src_c], rows, sem)
        pltpu.sync_copy(eag_h.at[pl.ds(ebase * EAW, CH * EAW)], eagv)
        cp.wait()
        def jb(j, c2):
          eagv[pl.ds(j * EAW, 16)] = rows[j, pl.ds(0, 16)] + eagv[pl.ds(j * EAW, 16)]
          return c2
        lax.fori_loop(0, CH, jb, 0)
        pltpu.sync_copy(eagv, out_h.at[pl.ds(ebase * EAW, CH * EAW)])
      return c
    lax.fori_loop(0, ipt, chunk, 0)

  return pl.kernel(
      body,
      out_type=jax.ShapeDtypeStruct((ne * EAW,), f32),
      mesh=_mesh(),
      compiler_params=_SC_PARAMS,
      scratch_types=[
          pltpu.VMEM((CH,), i32),
          pltpu.VMEM((CH, D), f32),
          pltpu.VMEM((CH * EAW,), f32),
          pltpu.SemaphoreType.DMA,
      ],
  )(src, td_tab, eag_flat)


def _mm(x, w, bias=None, relu=False, x2=None):
  m, kk = x.shape
  n = w.shape[1]
  bm = 512
  grid = m // bm
  have_b = bias is not None
  have_x2 = x2 is not None

  def body(*refs):
    idx = 0
    x_ref = refs[idx]; idx += 1
    if have_x2:
      x2_ref = refs[idx]; idx += 1
    w_ref = refs[idx]; idx += 1
    if have_b:
      b_ref = refs[idx]; idx += 1
    o_ref = refs[idx]
    xv = x_ref[...]
    if have_x2:
      xv = xv + x2_ref[...]
    acc = jnp.dot(xv, w_ref[...], preferred_element_type=f32)
    if have_b:
      acc = acc + b_ref[...]
    if relu:
      acc = jnp.maximum(acc, 0.0)
    o_ref[...] = acc

  in_specs = [pl.BlockSpec((bm, kk), lambda i: (i, 0))]
  args = [x]
  if have_x2:
    in_specs.append(pl.BlockSpec((bm, kk), lambda i: (i, 0)))
    args.append(x2)
  in_specs.append(pl.BlockSpec((kk, n), lambda i: (0, 0)))
  args.append(w)
  if have_b:
    in_specs.append(pl.BlockSpec((1, n), lambda i: (0, 0)))
    args.append(bias.reshape(1, n))
  return pl.pallas_call(
      body, grid=(grid,), in_specs=in_specs,
      out_specs=pl.BlockSpec((bm, n), lambda i: (i, 0)),
      out_shape=jax.ShapeDtypeStruct((m, n), f32))(*args)


def _mv(x, w, c):
  m, kk = x.shape
  bm = 512
  grid = m // bm

  def body(x_ref, w_ref, c_ref, o_ref):
    o_ref[...] = jnp.sum(x_ref[...] * w_ref[...], axis=1) + c_ref[...]

  return pl.pallas_call(
      body, grid=(grid,),
      in_specs=[pl.BlockSpec((bm, kk), lambda i: (i, 0)),
                pl.BlockSpec((1, kk), lambda i: (0, 0)),
                pl.BlockSpec((1,), lambda i: (0,))],
      out_specs=pl.BlockSpec((bm,), lambda i: (i,)),
      out_shape=jax.ShapeDtypeStruct((m,), f32))(
          x, w.reshape(1, kk), jnp.asarray(c, f32).reshape(1))


def _combine_s(a0, a1, a2, a3, r_os, r_ss, b_os, b_ss, relu):
  bm = 512
  grid = NSP // bm

  def body(a0r, a1r, a2r, a3r, ror, rsr, bor, bsr, o_ref):
    satt_os = 1.0 - 1e-9 * ror[...]
    satt_ss = 1.0 - 1e-9 * rsr[...]
    acc = (a0r[...] + a1r[...] + a2r[...] + a3r[...]
           + satt_os[:, None] * bor[...] + satt_ss[:, None] * bsr[...])
    if relu:
      acc = jnp.maximum(acc, 0.0)
    o_ref[...] = acc

  bs2 = pl.BlockSpec((bm, D), lambda i: (i, 0))
  bs1 = pl.BlockSpec((bm,), lambda i: (i,))
  bsb = pl.BlockSpec((1, D), lambda i: (0, 0))
  return pl.pallas_call(
      body, grid=(grid,),
      in_specs=[bs2, bs2, bs2, bs2, bs1, bs1, bsb, bsb],
      out_specs=bs2,
      out_shape=jax.ShapeDtypeStruct((NSP, D), f32))(
          a0, a1, a2, a3, r_os, r_ss,
          b_os.reshape(1, D), b_ss.reshape(1, D))


def _combine_o(o_mm, msg_mm, deg, b_o, b_so, relu):
  bm = 512
  grid = NSP // bm

  def body(omr, mmr, dgr, bor, bsr, o_ref):
    dg = dgr[...]
    acc = omr[...] + bor[...] + (mmr[...] + dg[:, None] * bsr[...]) / (dg[:, None] + 1.0)
    if relu:
      acc = jnp.maximum(acc, 0.0)
    o_ref[...] = acc

  bs2 = pl.BlockSpec((bm, D), lambda i: (i, 0))
  bs1 = pl.BlockSpec((bm,), lambda i: (i,))
  bsb = pl.BlockSpec((1, D), lambda i: (0, 0))
  return pl.pallas_call(
      body, grid=(grid,),
      in_specs=[bs2, bs2, bs1, bsb, bsb],
      out_specs=bs2,
      out_shape=jax.ShapeDtypeStruct((NSP, D), f32))(
          o_mm, msg_mm, deg, b_o.reshape(1, D), b_so.reshape(1, D))


def kernel(s_feat, o_feat, os_edge_attr, ss_edge_attr, params,
           os_src, os_dst, ss_src, ss_dst):
  ns, _ = s_feat.shape
  no, _ = o_feat.shape
  ne = os_src.shape[0]
  n_layers = len(params)

  pad_n = lambda x: jnp.pad(x, ((0, NSP - x.shape[0]), (0, 0)))
  s_cur = pad_n(s_feat.astype(f32))
  o_cur = pad_n(o_feat.astype(f32))
  os_src = os_src.astype(i32)
  os_dst = os_dst.astype(i32)
  ss_src = ss_src.astype(i32)
  ss_dst = ss_dst.astype(i32)
  os_ea = os_edge_attr.astype(f32)
  ss_ea = ss_edge_attr.astype(f32)

  deg = _k_deg(os_src, ne=ne)

  s_hid = o_hid = delta16 = None
  for li, p in enumerate(params):
    od = p['W_o'].shape[0]
    sd = p['W_so'].shape[0]
    w_os_top, w_os_bot = p['W_os'][:od], p['W_os'][od:]
    w_ss_top, w_ss_bot = p['W_ss'][:sd], p['W_ss'][sd:]
    wq_os = w_os_bot @ p['a_os']
    wq_ss = w_ss_bot @ p['a_ss']
    c_os = jnp.dot(p['b_os'], p['a_os'])
    c_ss = jnp.dot(p['b_ss'], p['a_ss'])

    p_o = _mm(o_cur, w_os_top)
    p_s = _mm(s_cur, w_ss_top)
    po = _mv(p_o, p['a_os'], 0.0)
    ps = _mv(p_s, p['a_ss'], 0.0)
    q_os = _mv(os_ea, wq_os, c_os)
    q_ss = _mv(ss_ea, wq_ss, c_ss)

    m_os, r_os, att_os = _k_soft(po, q_os, os_src, os_dst, ne=ne)
    m_ss, r_ss, att_ss = _k_soft(ps, q_ss, ss_src, ss_dst, ne=ne)

    ea2_os = _mm(os_ea, w_os_bot)
    ea2_ss = _mm(ss_ea, w_ss_bot)
    aggp_os = _k_heavy(os_src, os_dst, att_os, p_o, ea2_os, ne=ne)
    aggp_ss = _k_heavy(ss_src, ss_dst, att_ss, p_s, ea2_ss, ne=ne)
    msgp = _k_msg(os_dst, os_src, s_cur, ne=ne)

    msg_mm = _mm(msgp[0], p['W_so'], x2=msgp[1])
    o_mm = _mm(o_cur, p['W_o'])

    last = li == n_layers - 1
    s_hid = _combine_s(aggp_os[0], aggp_os[1], aggp_ss[0], aggp_ss[1],
                       r_os, r_ss, p['b_os'], p['b_ss'],
                       relu=not last)
    o_hid = _combine_o(o_mm, msg_mm, deg, p['b_o'], p['b_so'], relu=not last)

    if last:
      wd128 = jnp.pad(p['W_delta'], ((0, 0), (0, D - p['W_delta'].shape[1])))
      td128 = _mm(p_s, wd128)
      g16 = w_ss_bot @ wd128[:, :EAW]
      cvec16 = p['b_ss'] @ wd128[:, :EAW] + jnp.pad(
          p['b_delta'], (0, EAW - p['b_delta'].shape[0]))
      eag = _mm(ss_ea, g16, bias=cvec16)
      delta16 = _k_delta(ss_src, td128, eag.reshape(-1), ne=ne).reshape(ne, EAW)

    s_cur, o_cur = s_hid, o_hid

  logits = _k_logits(os_src, os_dst, s_hid, o_hid, ne=ne)
  return (logits, delta16[:, :p['W_delta'].shape[1]])


# speedup vs baseline: 4.0644x; 1.0117x over previous
import functools

import jax
import jax.numpy as jnp
from jax import lax
from jax.experimental import pallas as pl
from jax.experimental.pallas import tpu as pltpu
from jax.experimental.pallas import tpu_sc as plsc

NSP = 10240
D = 128
EAW = 16
NC = 2
NSC = 16
SL = NSP // NSC
CH = 128
NEG = -3.0e38

f32 = jnp.float32
i32 = jnp.int32


def _mesh():
  return plsc.VectorSubcoreMesh(core_axis_name="c", subcore_axis_name="s",
                                num_cores=NC, num_subcores=NSC)


_SC_PARAMS = pltpu.CompilerParams(needs_layout_passes=False)


def _leaky(x):
  return jnp.where(x >= 0, x, 0.2 * x)


def _lane_iota():
  return lax.iota(i32, 16)


def _seg_max(arr, idx16, val16):
  def body(_):
    g = plsc.load_gather(arr, [idx16])
    need = val16 > g
    plsc.store_scatter(arr, [idx16], jnp.maximum(g, val16), mask=need)
    return jnp.any(need)
  lax.while_loop(lambda c: c, body, jnp.any(val16 > plsc.load_gather(arr, [idx16])))


def _seg_add(arr, aux, idx16, val16):
  lid = _lane_iota()
  def cond(pending):
    return jnp.any(pending)
  def body(pending):
    plsc.store_scatter(aux, [idx16], lid, mask=pending)
    win = (plsc.load_gather(aux, [idx16]) == lid) & pending
    g = plsc.load_gather(arr, [idx16])
    plsc.store_scatter(arr, [idx16], g + val16, mask=win)
    return pending & jnp.logical_not(win)
  lax.while_loop(cond, body, jnp.ones((16,), jnp.bool_))


def _fill1d(ref, n, value):
  def b(i, c):
    ref[pl.ds(i * 16, 16)] = jnp.full((16,), value, f32)
    return c
  lax.fori_loop(0, n // 16, b, 0)


def _merge_tiles(part_sh, macc, mtmp, moff, op):
  pltpu.sync_copy(part_sh.at[0, pl.ds(moff, SL)], macc)
  def mb(j, c):
    pltpu.sync_copy(part_sh.at[j, pl.ds(moff, SL)], mtmp)
    def vb(i, c2):
      a = macc[pl.ds(i * 16, 16)]
      b = mtmp[pl.ds(i * 16, 16)]
      macc[pl.ds(i * 16, 16)] = op(a, b)
      return c2
    lax.fori_loop(0, SL // 16, vb, 0)
    return c
  lax.fori_loop(1, NSC, mb, 0)


@functools.partial(jax.jit, static_argnames=("ne",))
def _k_soft(po, q, src, dst, ne):
  epw = ne // NSC
  chk = 2000
  nchk = epw // chk

  def body(po_h, q_h, src_h, dst_h, m_h, r_h, att_h,
           po_v, dst_v, e_v, acc_v, aux_v, m_v, r_v, srcc, qc, macc, mtmp,
           part_sh, m_sh, r_sh):
    cid = lax.axis_index("c")
    sid = lax.axis_index("s")
    base = sid * epw
    moff = sid * SL
    pltpu.sync_copy(po_h, po_v)
    pltpu.sync_copy(dst_h.at[pl.ds(base, epw)], dst_v)
    _fill1d(acc_v, NSP, NEG)

    def chunk(k, c):
      cb = base + k * chk
      pltpu.sync_copy(src_h.at[pl.ds(cb, chk)], srcc)
      pltpu.sync_copy(q_h.at[pl.ds(cb, chk)], qc)
      def vb(i, c2):
        s16 = srcc[pl.ds(i * 16, 16)]
        q16 = qc[pl.ds(i * 16, 16)]
        e16 = _leaky(plsc.load_gather(po_v, [s16]) + q16)
        off = k * chk + i * 16
        e_v[pl.ds(off, 16)] = e16
        d16 = dst_v[pl.ds(off, 16)]
        _seg_max(acc_v, d16, e16)
        return c2
      lax.fori_loop(0, chk // 16, vb, 0)
      return c
    lax.fori_loop(0, nchk, chunk, 0)

    pltpu.sync_copy(acc_v, part_sh.at[sid])
    plsc.subcore_barrier()
    _merge_tiles(part_sh, macc, mtmp, moff, jnp.maximum)
    pltpu.sync_copy(macc, m_sh.at[pl.ds(moff, SL)])
    @pl.when(cid == 0)
    def _():
      pltpu.sync_copy(macc, m_h.at[pl.ds(moff, SL)])
    plsc.subcore_barrier()
    pltpu.sync_copy(m_sh, m_v)
    _fill1d(acc_v, NSP, 0.0)

    def vb2(i, c):
      e16 = e_v[pl.ds(i * 16, 16)]
      d16 = dst_v[pl.ds(i * 16, 16)]
      ex = jnp.exp(e16 - plsc.load_gather(m_v, [d16]))
      plsc.addupdate_scatter(acc_v, [d16], ex)
      return c
    lax.fori_loop(0, epw // 16, vb2, 0)

    pltpu.sync_copy(acc_v, part_sh.at[sid])
    plsc.subcore_barrier()
    _merge_tiles(part_sh, macc, mtmp, moff, jnp.add)
    def vb3(i, c):
      macc[pl.ds(i * 16, 16)] = 1.0 / (macc[pl.ds(i * 16, 16)] + 1e-9)
      return c
    lax.fori_loop(0, SL // 16, vb3, 0)
    pltpu.sync_copy(macc, r_sh.at[pl.ds(moff, SL)])
    @pl.when(cid == 0)
    def _():
      pltpu.sync_copy(macc, r_h.at[pl.ds(moff, SL)])
    plsc.subcore_barrier()
    pltpu.sync_copy(r_sh, r_v)

    def vb4(i, c):
      e16 = e_v[pl.ds(i * 16, 16)]
      d16 = dst_v[pl.ds(i * 16, 16)]
      mg = plsc.load_gather(m_v, [d16])
      rg = plsc.load_gather(r_v, [d16])
      e_v[pl.ds(i * 16, 16)] = jnp.exp(e16 - mg) * rg
      return c
    lax.fori_loop(0, epw // 16, vb4, 0)
    @pl.when(cid == 0)
    def _():
      pltpu.sync_copy(e_v, att_h.at[pl.ds(base, epw)])

  return pl.kernel(
      body,
      out_type=(jax.ShapeDtypeStruct((NSP,), f32),
                jax.ShapeDtypeStruct((NSP,), f32),
                jax.ShapeDtypeStruct((ne,), f32)),
      mesh=_mesh(),
      compiler_params=_SC_PARAMS,
      scratch_types=[
          pltpu.VMEM((NSP,), f32),
          pltpu.VMEM((epw,), i32),
          pltpu.VMEM((epw,), f32),
          pltpu.VMEM((NSP,), f32),
          pltpu.VMEM((NSP,), i32),
          pltpu.VMEM((NSP,), f32),
          pltpu.VMEM((NSP,), f32),
          pltpu.VMEM((chk,), i32),
          pltpu.VMEM((chk,), f32),
          pltpu.VMEM((SL,), f32),
          pltpu.VMEM((SL,), f32),
          pltpu.VMEM_SHARED((NSC, NSP), f32),
          pltpu.VMEM_SHARED((NSP,), f32),
          pltpu.VMEM_SHARED((NSP,), f32),
      ],
  )(po, q, src, dst)


@functools.partial(jax.jit, static_argnames=("ne",))
def _k_deg(src, ne):
  epw = ne // NSC

  def body(src_h, deg_h, src_v, acc_v, aux_v, macc, mtmp, part_sh):
    cid = lax.axis_index("c")
    sid = lax.axis_index("s")
    moff = sid * SL
    pltpu.sync_copy(src_h.at[pl.ds(sid * epw, epw)], src_v)
    _fill1d(acc_v, NSP, 0.0)
    ones = jnp.ones((16,), f32)
    def vb(i, c):
      s16 = src_v[pl.ds(i * 16, 16)]
      plsc.addupdate_scatter(acc_v, [s16], ones)
      return c
    lax.fori_loop(0, epw // 16, vb, 0)
    pltpu.sync_copy(acc_v, part_sh.at[sid])
    plsc.subcore_barrier()
    _merge_tiles(part_sh, macc, mtmp, moff, jnp.add)
    @pl.when(cid == 0)
    def _():
      pltpu.sync_copy(macc, deg_h.at[pl.ds(moff, SL)])

  return pl.kernel(
      body,
      out_type=jax.ShapeDtypeStruct((NSP,), f32),
      mesh=_mesh(),
      compiler_params=_SC_PARAMS,
      scratch_types=[
          pltpu.VMEM((epw,), i32),
          pltpu.VMEM((NSP,), f32),
          pltpu.VMEM((NSP,), i32),
          pltpu.VMEM((SL,), f32),
          pltpu.VMEM((SL,), f32),
          pltpu.VMEM_SHARED((NSC, NSP), f32),
      ],
  )(src)


@functools.partial(jax.jit, static_argnames=("ne",))
def _k_heavy(src, dst, att, p_tab, ea2, ne):
  chh = 64
  epc = ne // NC
  nch = epc // chh
  ipt = (nch + NSC - 1) // NSC
  npair = (ipt + 1) // 2

  def body(src_h, dst_h, att_h, p_h, ea2_h, aggp_h, *scr):
    (src_c, dstw, att_c, rows, erows, sems, esems, acc_sh) = (
        scr[0:2], scr[2:4], scr[4:6], scr[6:8], scr[8:10], scr[10:12],
        scr[12:14], scr[14])
    cid = lax.axis_index("c")
    sid = lax.axis_index("s")
    moff = sid * SL

    def zb(j, c):
      for f in range(D // 16):
        rows[0][j, pl.ds(f * 16, 16)] = jnp.zeros((16,), f32)
      return c
    lax.fori_loop(0, chh, zb, 0)
    for blk in range(SL // chh):
      pltpu.sync_copy(rows[0], acc_sh.at[pl.ds(moff + blk * chh, chh)])
    plsc.subcore_barrier()

    def issue(b, k):
      @pl.when(k < nch)
      def _():
        ebase = cid * epc + k * chh
        pltpu.sync_copy(src_h.at[pl.ds(ebase, chh)], src_c[b])
        pltpu.sync_copy(dst_h.at[pl.ds(ebase, chh)], dstw[b].at[0])
        pltpu.sync_copy(att_h.at[pl.ds(ebase, chh)], att_c[b])
        pltpu.async_copy(p_h.at[src_c[b]], rows[b], sems[b])
        pltpu.async_copy(ea2_h.at[pl.ds(ebase, chh)], erows[b], esems[b])

    def finish(b, k):
      @pl.when(k < nch)
      def _():
        pltpu.make_async_copy(p_h.at[src_c[b]], rows[b], sems[b]).wait()
        pltpu.make_async_copy(ea2_h.at[pl.ds(0, chh)], erows[b], esems[b]).wait()
        def sb(j, c2):
          ab16 = plsc.load_gather(att_c[b], [jnp.full((16,), j, i32)])
          for f in range(D // 16):
            rows[b][j, pl.ds(f * 16, 16)] = (
                rows[b][j, pl.ds(f * 16, 16)]
                + erows[b][j, pl.ds(f * 16, 16)]) * ab16
          return c2
        lax.fori_loop(0, chh, sb, 0)
        pltpu.sync_copy(rows[b], acc_sh.at[dstw[b].at[0]], add=True)

    issue(0, sid)
    def pair(i, c):
      k0 = sid + (2 * i) * NSC
      k1 = sid + (2 * i + 1) * NSC
      issue(1, k1)
      finish(0, k0)
      issue(0, sid + (2 * i + 2) * NSC)
      finish(1, k1)
      return c
    lax.fori_loop(0, npair, pair, 0)

    plsc.subcore_barrier()
    pltpu.sync_copy(acc_sh.at[pl.ds(moff, SL)], aggp_h.at[cid, pl.ds(moff, SL)])

  return pl.kernel(
      body,
      out_type=jax.ShapeDtypeStruct((NC, NSP, D), f32),
      mesh=_mesh(),
      compiler_params=_SC_PARAMS,
      scratch_types=[
          pltpu.VMEM((chh,), i32),
          pltpu.VMEM((chh,), i32),
          pltpu.VMEM((1, chh), i32),
          pltpu.VMEM((1, chh), i32),
          pltpu.VMEM((chh,), f32),
          pltpu.VMEM((chh,), f32),
          pltpu.VMEM((chh, D), f32),
          pltpu.VMEM((chh, D), f32),
          pltpu.VMEM((chh, D), f32),
          pltpu.VMEM((chh, D), f32),
          pltpu.SemaphoreType.DMA,
          pltpu.SemaphoreType.DMA,
          pltpu.SemaphoreType.DMA,
          pltpu.SemaphoreType.DMA,
          pltpu.VMEM_SHARED((NSP, D), f32),
      ],
  )(src, dst, att, p_tab, ea2)


@functools.partial(jax.jit, static_argnames=("ne",))
def _k_msg(gidx, sidx, tab, ne):
  epc = ne // NC
  nch = epc // CH
  ipt = (nch + NSC - 1) // NSC

  def body(g_h, s_h, tab_h, out_h, g_c, sw, rows, sem, acc_sh):
    cid = lax.axis_index("c")
    sid = lax.axis_index("s")
    moff = sid * SL
    def zb(j, c):
      for f in range(D // 16):
        rows[j, pl.ds(f * 16, 16)] = jnp.zeros((16,), f32)
      return c
    lax.fori_loop(0, CH, zb, 0)
    for blk in range(SL // CH):
      pltpu.sync_copy(rows, acc_sh.at[pl.ds(moff + blk * CH, CH)])
    plsc.subcore_barrier()

    def chunk(i, c):
      k = sid + i * NSC
      @pl.when(k < nch)
      def _():
        ebase = cid * epc + k * CH
        pltpu.sync_copy(g_h.at[pl.ds(ebase, CH)], g_c)
        pltpu.sync_copy(s_h.at[pl.ds(ebase, CH)], sw.at[0])
        pltpu.async_copy(tab_h.at[g_c], rows, sem).wait()
        pltpu.sync_copy(rows, acc_sh.at[sw.at[0]], add=True)
      return c
    lax.fori_loop(0, ipt, chunk, 0)

    plsc.subcore_barrier()
    pltpu.sync_copy(acc_sh.at[pl.ds(moff, SL)], out_h.at[cid, pl.ds(moff, SL)])

  return pl.kernel(
      body,
      out_type=jax.ShapeDtypeStruct((NC, NSP, D), f32),
      mesh=_mesh(),
      compiler_params=_SC_PARAMS,
      scratch_types=[
          pltpu.VMEM((CH,), i32),
          pltpu.VMEM((1, CH), i32),
          pltpu.VMEM((CH, D), f32),
          pltpu.SemaphoreType.DMA,
          pltpu.VMEM_SHARED((NSP, D), f32),
      ],
  )(gidx, sidx, tab)


@functools.partial(jax.jit, static_argnames=("ne",))
def _k_logits(src, dst, s_tab, o_tab, ne):
  epc = ne // NC
  nch = epc // CH
  ipt = (nch + NSC - 1) // NSC

  def body(src_h, dst_h, s_h, o_h, out_h, src_c, dst_c, srows, orows,
           lg_c, sem1, sem2):
    cid = lax.axis_index("c")
    sid = lax.axis_index("s")
    def chunk(i, c):
      k = sid + i * NSC
      @pl.when(k < nch)
      def _():
        ebase = cid * epc + k * CH
        pltpu.sync_copy(src_h.at[pl.ds(ebase, CH)], src_c)
        pltpu.sync_copy(dst_h.at[pl.ds(ebase, CH)], dst_c)
        cp1 = pltpu.async_copy(o_h.at[src_c], orows, sem1)
        cp2 = pltpu.async_copy(s_h.at[dst_c], srows, sem2)
        cp1.wait()
        cp2.wait()
        lid = _lane_iota()
        def gb(g, c2):
          def jb(jj, out16):
            j = g * 16 + jj
            acc = srows[j, pl.ds(0, 16)] * orows[j, pl.ds(0, 16)]
            for f in range(1, D // 16):
              acc = acc + srows[j, pl.ds(f * 16, 16)] * orows[j, pl.ds(f * 16, 16)]
            dot = jnp.sum(acc)
            return jnp.where(lid == jj, dot, out16)
          out16 = lax.fori_loop(0, 16, jb, jnp.zeros((16,), f32))
          lg_c[pl.ds(g * 16, 16)] = out16
          return c2
        lax.fori_loop(0, CH // 16, gb, 0)
        pltpu.sync_copy(lg_c, out_h.at[pl.ds(ebase, CH)])
      return c
    lax.fori_loop(0, ipt, chunk, 0)

  return pl.kernel(
      body,
      out_type=jax.ShapeDtypeStruct((ne,), f32),
      mesh=_mesh(),
      compiler_params=_SC_PARAMS,
      scratch_types=[
          pltpu.VMEM((CH,), i32),
          pltpu.VMEM((CH,), i32),
          pltpu.VMEM((CH, D), f32),
          pltpu.VMEM((CH, D), f32),
          pltpu.VMEM((CH,), f32),
          pltpu.SemaphoreType.DMA,
          pltpu.SemaphoreType.DMA,
      ],
  )(src, dst, s_tab, o_tab)


@functools.partial(jax.jit, static_argnames=("ne",))
def _k_delta(src, td_tab, eag_flat, ne):
  epc = ne // NC
  nch = epc // CH
  ipt = (nch + NSC - 1) // NSC

  def body(src_h, td_h, eag_h, out_h, src_c, rows, eagv, sem):
    cid = lax.axis_index("c")
    sid = lax.axis_index("s")
    def chunk(i, c):
      k = sid + i * NSC
      @pl.when(k < nch)
      def _():
        ebase = cid * epc + k * CH
        pltpu.sync_copy(src_h.at[pl.ds(ebase, CH)], src_c)
        cp = pltpu.async_copy(td_h.at[src_c], rows, sem)
        pltpu.sync_copy(eag_h.at[pl.ds(ebase * EAW, CH * EAW)], eagv)
        cp.wait()
        def jb(j, c2):
          eagv[pl.ds(j * EAW, 16)] = rows[j, pl.ds(0, 16)] + eagv[pl.ds(j * EAW, 16)]
          return c2
        lax.fori_loop(0, CH, jb, 0)
        pltpu.sync_copy(eagv, out_h.at[pl.ds(ebase * EAW, CH * EAW)])
      return c
    lax.fori_loop(0, ipt, chunk, 0)

  return pl.kernel(
      body,
      out_type=jax.ShapeDtypeStruct((ne * EAW,), f32),
      mesh=_mesh(),
      compiler_params=_SC_PARAMS,
      scratch_types=[
          pltpu.VMEM((CH,), i32),
          pltpu.VMEM((CH, D), f32),
          pltpu.VMEM((CH * EAW,), f32),
          pltpu.SemaphoreType.DMA,
      ],
  )(src, td_tab, eag_flat)


def _mm(x, w, bias=None, relu=False, x2=None):
  m, kk = x.shape
  n = w.shape[1]
  bm = 512
  grid = m // bm
  have_b = bias is not None
  have_x2 = x2 is not None

  def body(*refs):
    idx = 0
    x_ref = refs[idx]; idx += 1
    if have_x2:
      x2_ref = refs[idx]; idx += 1
    w_ref = refs[idx]; idx += 1
    if have_b:
      b_ref = refs[idx]; idx += 1
    o_ref = refs[idx]
    xv = x_ref[...]
    if have_x2:
      xv = xv + x2_ref[...]
    acc = jnp.dot(xv, w_ref[...], preferred_element_type=f32)
    if have_b:
      acc = acc + b_ref[...]
    if relu:
      acc = jnp.maximum(acc, 0.0)
    o_ref[...] = acc

  in_specs = [pl.BlockSpec((bm, kk), lambda i: (i, 0))]
  args = [x]
  if have_x2:
    in_specs.append(pl.BlockSpec((bm, kk), lambda i: (i, 0)))
    args.append(x2)
  in_specs.append(pl.BlockSpec((kk, n), lambda i: (0, 0)))
  args.append(w)
  if have_b:
    in_specs.append(pl.BlockSpec((1, n), lambda i: (0, 0)))
    args.append(bias.reshape(1, n))
  return pl.pallas_call(
      body, grid=(grid,), in_specs=in_specs,
      out_specs=pl.BlockSpec((bm, n), lambda i: (i, 0)),
      out_shape=jax.ShapeDtypeStruct((m, n), f32))(*args)


def _mv(x, w, c):
  m, kk = x.shape
  bm = 512
  grid = m // bm

  def body(x_ref, w_ref, c_ref, o_ref):
    o_ref[...] = jnp.sum(x_ref[...] * w_ref[...], axis=1) + c_ref[...]

  return pl.pallas_call(
      body, grid=(grid,),
      in_specs=[pl.BlockSpec((bm, kk), lambda i: (i, 0)),
                pl.BlockSpec((1, kk), lambda i: (0, 0)),
                pl.BlockSpec((1,), lambda i: (0,))],
      out_specs=pl.BlockSpec((bm,), lambda i: (i,)),
      out_shape=jax.ShapeDtypeStruct((m,), f32))(
          x, w.reshape(1, kk), jnp.asarray(c, f32).reshape(1))


def _combine_s(a0, a1, a2, a3, r_os, r_ss, b_os, b_ss, relu):
  bm = 512
  grid = NSP // bm

  def body(a0r, a1r, a2r, a3r, ror, rsr, bor, bsr, o_ref):
    satt_os = 1.0 - 1e-9 * ror[...]
    satt_ss = 1.0 - 1e-9 * rsr[...]
    acc = (a0r[...] + a1r[...] + a2r[...] + a3r[...]
           + satt_os[:, None] * bor[...] + satt_ss[:, None] * bsr[...])
    if relu:
      acc = jnp.maximum(acc, 0.0)
    o_ref[...] = acc

  bs2 = pl.BlockSpec((bm, D), lambda i: (i, 0))
  bs1 = pl.BlockSpec((bm,), lambda i: (i,))
  bsb = pl.BlockSpec((1, D), lambda i: (0, 0))
  return pl.pallas_call(
      body, grid=(grid,),
      in_specs=[bs2, bs2, bs2, bs2, bs1, bs1, bsb, bsb],
      out_specs=bs2,
      out_shape=jax.ShapeDtypeStruct((NSP, D), f32))(
          a0, a1, a2, a3, r_os, r_ss,
          b_os.reshape(1, D), b_ss.reshape(1, D))


def _combine_o(o_mm, msg_mm, deg, b_o, b_so, relu):
  bm = 512
  grid = NSP // bm

  def body(omr, mmr, dgr, bor, bsr, o_ref):
    dg = dgr[...]
    acc = omr[...] + bor[...] + (mmr[...] + dg[:, None] * bsr[...]) / (dg[:, None] + 1.0)
    if relu:
      acc = jnp.maximum(acc, 0.0)
    o_ref[...] = acc

  bs2 = pl.BlockSpec((bm, D), lambda i: (i, 0))
  bs1 = pl.BlockSpec((bm,), lambda i: (i,))
  bsb = pl.BlockSpec((1, D), lambda i: (0, 0))
  return pl.pallas_call(
      body, grid=(grid,),
      in_specs=[bs2, bs2, bs1, bsb, bsb],
      out_specs=bs2,
      out_shape=jax.ShapeDtypeStruct((NSP, D), f32))(
          o_mm, msg_mm, deg, b_o.reshape(1, D), b_so.reshape(1, D))


def kernel(s_feat, o_feat, os_edge_attr, ss_edge_attr, params,
           os_src, os_dst, ss_src, ss_dst):
  ns, _ = s_feat.shape
  no, _ = o_feat.shape
  ne = os_src.shape[0]
  n_layers = len(params)

  pad_n = lambda x: jnp.pad(x, ((0, NSP - x.shape[0]), (0, 0)))
  s_cur = pad_n(s_feat.astype(f32))
  o_cur = pad_n(o_feat.astype(f32))
  os_src = os_src.astype(i32)
  os_dst = os_dst.astype(i32)
  ss_src = ss_src.astype(i32)
  ss_dst = ss_dst.astype(i32)
  os_ea = os_edge_attr.astype(f32)
  ss_ea = ss_edge_attr.astype(f32)

  deg = _k_deg(os_src, ne=ne)

  s_hid = o_hid = delta16 = None
  for li, p in enumerate(params):
    od = p['W_o'].shape[0]
    sd = p['W_so'].shape[0]
    w_os_top, w_os_bot = p['W_os'][:od], p['W_os'][od:]
    w_ss_top, w_ss_bot = p['W_ss'][:sd], p['W_ss'][sd:]
    wq_os = w_os_bot @ p['a_os']
    wq_ss = w_ss_bot @ p['a_ss']
    c_os = jnp.dot(p['b_os'], p['a_os'])
    c_ss = jnp.dot(p['b_ss'], p['a_ss'])

    p_o = _mm(o_cur, w_os_top)
    p_s = _mm(s_cur, w_ss_top)
    po = _mv(p_o, p['a_os'], 0.0)
    ps = _mv(p_s, p['a_ss'], 0.0)
    q_os = _mv(os_ea, wq_os, c_os)
    q_ss = _mv(ss_ea, wq_ss, c_ss)

    m_os, r_os, att_os = _k_soft(po, q_os, os_src, os_dst, ne=ne)
    m_ss, r_ss, att_ss = _k_soft(ps, q_ss, ss_src, ss_dst, ne=ne)

    ea2_os = _mm(os_ea, w_os_bot)
    ea2_ss = _mm(ss_ea, w_ss_bot)
    aggp_os = _k_heavy(os_src, os_dst, att_os, p_o, ea2_os, ne=ne)
    aggp_ss = _k_heavy(ss_src, ss_dst, att_ss, p_s, ea2_ss, ne=ne)
    msgp = _k_msg(os_dst, os_src, s_cur, ne=ne)

    msg_mm = _mm(msgp[0], p['W_so'], x2=msgp[1])
    o_mm = _mm(o_cur, p['W_o'])

    last = li == n_layers - 1
    s_hid = _combine_s(aggp_os[0], aggp_os[1], aggp_ss[0], aggp_ss[1],
                       r_os, r_ss, p['b_os'], p['b_ss'],
                       relu=not last)
    o_hid = _combine_o(o_mm, msg_mm, deg, p['b_o'], p['b_so'], relu=not last)

    if last:
      wd128 = jnp.pad(p['W_delta'], ((0, 0), (0, D - p['W_delta'].shape[1])))
      td128 = _mm(p_s, wd128)
      g16 = w_ss_bot @ wd128[:, :EAW]
      cvec16 = p['b_ss'] @ wd128[:, :EAW] + jnp.pad(
          p['b_delta'], (0, EAW - p['b_delta'].shape[0]))
      eag = _mm(ss_ea, g16, bias=cvec16)
      delta16 = _k_delta(ss_src, td128, eag.reshape(-1), ne=ne).reshape(ne, EAW)

    s_cur, o_cur = s_hid, o_hid

  logits = _k_logits(os_src, os_dst, s_hid, o_hid, ne=ne)
  return (logits, delta16[:, :p['W_delta'].shape[1]])
